# baseline (speedup 1.0000x reference)
import jax
import jax.numpy as jnp
from jax.experimental import pallas as pl
from jax.experimental.pallas import tpu as pltpu

_B = 8
_A = 70400
_TA = 1760
_NB = _A // _TA

_SIGMA2 = 9.0
_ALPHA = 0.25
_LOC_W = 2.0
_CLS_W = 1.0
_DIR_W = 0.2
_TWO_PI = 6.283185307179586
_PI = 3.141592653589793


def _smooth_l1(d):
    ad = jnp.abs(d)
    return jnp.where(ad < (1.0 / _SIGMA2), 0.5 * _SIGMA2 * d * d,
                     ad - 0.5 / _SIGMA2)


def _loss_kernel(box_ref, cls_ref, dir_ref, reg_ref, anc_ref, lab_ref,
                 out_ref, acc_ref):
    b = pl.program_id(0)
    n = pl.program_id(1)

    lab = lab_ref[0]
    pos = (lab > 0).astype(jnp.float32)
    neg = (lab == 0).astype(jnp.float32)

    box = box_ref[0]
    reg = reg_ref[0]

    d = box - reg
    lane = jax.lax.broadcasted_iota(jnp.int32, (_TA, 7), 1)
    l_main = jnp.where(lane < 6, _smooth_l1(d), 0.0).sum(axis=1, keepdims=True)
    d6 = jnp.sin(box[:, 6:7] - reg[:, 6:7])
    loc_s = ((l_main + _smooth_l1(d6)) * pos).sum()

    c = cls_ref[0]
    t = (jnp.where(lab >= 0, lab, 0) == 1).astype(jnp.float32)
    ce = jnp.maximum(c, 0.0) - c * t + jnp.log1p(jnp.exp(-jnp.abs(c)))
    p = jax.nn.sigmoid(c)
    pt = t * p + (1.0 - t) * (1.0 - p)
    aw = t * _ALPHA + (1.0 - t) * (1.0 - _ALPHA)
    cw = neg + pos
    cls_s = ((1.0 - pt) * (1.0 - pt) * aw * ce * cw).sum()

    rot = reg[:, 6:7] + anc_ref[0][:, 6:7]
    off = rot - jnp.floor(rot / _TWO_PI) * _TWO_PI
    tgt = jnp.clip(jnp.floor(off / _PI), 0.0, 1.0)
    dirs = dir_ref[0]
    d0 = dirs[:, 0:1]
    d1 = dirs[:, 1:2]
    lse = jnp.maximum(d0, d1) + jnp.log1p(jnp.exp(-jnp.abs(d0 - d1)))
    dsel = jnp.where(tgt > 0.5, d1, d0)
    dir_s = ((lse - dsel) * pos).sum()

    npos = pos.sum()

    @pl.when(n == 0)
    def _():
        acc_ref[b, 0] = loc_s
        acc_ref[b, 1] = cls_s
        acc_ref[b, 2] = dir_s
        acc_ref[b, 3] = npos

    @pl.when(n > 0)
    def _():
        acc_ref[b, 0] += loc_s
        acc_ref[b, 1] += cls_s
        acc_ref[b, 2] += dir_s
        acc_ref[b, 3] += npos

    @pl.when((b == _B - 1) & (n == _NB - 1))
    def _():
        tot = 0.0
        for bb in range(_B):
            norm = jnp.maximum(acc_ref[bb, 3], 1.0)
            tot += (_LOC_W * acc_ref[bb, 0] + _CLS_W * acc_ref[bb, 1]
                    + _DIR_W * acc_ref[bb, 2]) / norm
        out_ref[...] = jnp.reshape(tot / _B, (1, 1))


def kernel(box_preds, cls_preds, dir_cls_preds, reg_targets, anchors, labels):
    labels3 = labels[..., None]
    out = pl.pallas_call(
        _loss_kernel,
        grid=(_B, _NB),
        in_specs=[
            pl.BlockSpec((1, _TA, 7), lambda b, n: (b, n, 0)),
            pl.BlockSpec((1, _TA, 1), lambda b, n: (b, n, 0)),
            pl.BlockSpec((1, _TA, 2), lambda b, n: (b, n, 0)),
            pl.BlockSpec((1, _TA, 7), lambda b, n: (b, n, 0)),
            pl.BlockSpec((1, _TA, 7), lambda b, n: (b, n, 0)),
            pl.BlockSpec((1, _TA, 1), lambda b, n: (b, n, 0)),
        ],
        out_specs=pl.BlockSpec((1, 1), lambda b, n: (0, 0)),
        out_shape=jax.ShapeDtypeStruct((1, 1), jnp.float32),
        scratch_shapes=[pltpu.SMEM((_B, 4), jnp.float32)],
    )(box_preds, cls_preds, dir_cls_preds, reg_targets, anchors, labels3)
    return out[0, 0]


# speedup vs baseline: 2.5479x; 2.5479x over previous
import functools

import jax
import jax.numpy as jnp
from jax import lax
from jax.experimental import pallas as pl
from jax.experimental.pallas import tpu as pltpu
from jax.experimental.pallas import tpu_sc as plsc

_B = 8
_A = 70400
_NTEC = 32
_QPB = 4
_PER_TEC = _A // _QPB
_CHUNK = 880
_NCHUNK = _PER_TEC // _CHUNK
_NGROUP = _CHUNK // 16

_R = 1.0 / 9.0
_HR = 0.5 / 9.0
_TWO_PI = 6.283185307179586
_INV_2PI = 1.0 / _TWO_PI
_INV_PI = 1.0 / 3.141592653589793

_S = (0.9999845867744923, -0.16663258204297338, 0.008312382933816597,
      -0.00019316182196016648, 2.173210068101922e-06)
_L = (5.621958997561607e-07, 0.9999574870751128, -0.4992065685483709,
      0.3269731000160856, -0.22283625833236495, 0.13076503250835383,
      -0.05262485136951819, 0.01011908292798416)


def _take(x, idx):
    return lax.gather(
        x, idx[:, None],
        lax.GatherDimensionNumbers(offset_dims=(), collapsed_slice_dims=(0,),
                                   start_index_map=(0,)),
        slice_sizes=(1,),
        mode=lax.GatherScatterMode.PROMISE_IN_BOUNDS)


def _floor(x):
    f = x.astype(jnp.int32).astype(jnp.float32)
    return jnp.where(f > x, f - 1.0, f)


def _log1p_poly(u):
    r = jnp.full_like(u, _L[7])
    for c in (_L[6], _L[5], _L[4], _L[3], _L[2], _L[1], _L[0]):
        r = r * u + c
    return r


def _sin_poly(x):
    n = _floor(x * _INV_2PI + 0.5)
    y = x - n * _TWO_PI
    y2 = y * y
    r = jnp.full_like(y, _S[4])
    for c in (_S[3], _S[2], _S[1], _S[0]):
        r = r * y2 + c
    return y * r


def _sc_body(box_hbm, cls_hbm, dir_hbm, reg_hbm, anc_hbm, lab_hbm, out_hbm,
             box0, reg0, anc0, cls0, dir0, lab0,
             box1, reg1, anc1, cls1, dir1, lab1,
             out_scr, sem0, sem1):
    cid = lax.axis_index("c")
    sid = lax.axis_index("s")
    wid = cid * 16 + sid
    b = wid // _QPB
    q = wid % _QPB
    base_anchor = b * _A + q * _PER_TEC

    bufs0 = (box0, reg0, anc0, cls0, dir0, lab0)
    bufs1 = (box1, reg1, anc1, cls1, dir1, lab1)

    def _copies(chunk, bufs, sem):
        a0 = base_anchor + chunk * _CHUNK
        return (
            pltpu.make_async_copy(box_hbm.at[pl.ds(a0 * 7, _CHUNK * 7)],
                                  bufs[0], sem),
            pltpu.make_async_copy(reg_hbm.at[pl.ds(a0 * 7, _CHUNK * 7)],
                                  bufs[1], sem),
            pltpu.make_async_copy(anc_hbm.at[pl.ds(a0 * 7, _CHUNK * 7)],
                                  bufs[2], sem),
            pltpu.make_async_copy(cls_hbm.at[pl.ds(a0, _CHUNK)],
                                  bufs[3], sem),
            pltpu.make_async_copy(dir_hbm.at[pl.ds(a0 * 2, _CHUNK * 2)],
                                  bufs[4], sem),
            pltpu.make_async_copy(lab_hbm.at[pl.ds(a0, _CHUNK)],
                                  bufs[5], sem),
        )

    def _start(chunk, bufs, sem):
        for c in _copies(chunk, bufs, sem):
            c.start()

    def _wait(chunk, bufs, sem):
        for c in _copies(chunk, bufs, sem):
            c.wait()

    iota = lax.iota(jnp.int32, 16)
    idx_ks = []
    notch6_ks = []
    for k in range(7):
        n = iota + 16 * k
        dv = ((n.astype(jnp.float32) + 0.5) * (1.0 / 7.0)).astype(jnp.int32)
        rem = n - dv * 7
        idx_ks.append(dv)
        notch6_ks.append(jnp.where(rem == 6, 0.0, 1.0))
    sevens6 = iota * 7 + 6
    cpm_ks = []
    vm_ks = []
    for k in range(7):
        pm = sevens6 - 16 * k
        cpm_ks.append(jnp.clip(pm, 0, 15))
        vm_ks.append(jnp.where((pm >= 0) & (pm <= 15), 1.0, 0.0))
    lo8 = iota < 8
    e0_lo = jnp.clip(iota * 2, 0, 15)
    e0_hi = jnp.clip(iota * 2 - 16, 0, 15)
    e1_lo = jnp.clip(iota * 2 + 1, 0, 15)
    e1_hi = jnp.clip(iota * 2 - 15, 0, 15)

    def _compute(bufs):
        box_b, reg_b, anc_b, cls_b, dir_b, lab_b = bufs

        def body(g, accs):
            acc_loc, acc_cls, acc_dir, acc_np = accs
            b16 = g * 16
            b112 = g * 112
            b32 = g * 32

            lab = lab_b[pl.ds(b16, 16)]
            pos = jnp.where(lab > 0, 1.0, 0.0)
            cared = jnp.where(lab >= 0, 1.0, 0.0)

            ds = []
            rots = []
            for k in range(7):
                rg = reg_b[pl.ds(b112 + 16 * k, 16)]
                d = box_b[pl.ds(b112 + 16 * k, 16)] - rg
                ds.append(d)
                rots.append(rg + anc_b[pl.ds(b112 + 16 * k, 16)])
                ad = jnp.abs(d)
                sl1 = jnp.where(ad < _R, 4.5 * d * d, ad - _HR)
                posx = _take(pos, idx_ks[k])
                acc_loc = acc_loc + sl1 * notch6_ks[k] * posx

            d6raw = _take(ds[0], cpm_ks[0]) * vm_ks[0]
            rot = _take(rots[0], cpm_ks[0]) * vm_ks[0]
            for k in range(1, 7):
                d6raw = d6raw + _take(ds[k], cpm_ks[k]) * vm_ks[k]
                rot = rot + _take(rots[k], cpm_ks[k]) * vm_ks[k]
            d6 = _sin_poly(d6raw)
            ad6 = jnp.abs(d6)
            sl6 = jnp.where(ad6 < _R, 4.5 * d6 * d6, ad6 - _HR)
            acc_loc = acc_loc + sl6 * pos

            c = cls_b[pl.ds(b16, 16)]
            t = jnp.where(lab == 1, 1.0, 0.0)
            e = jnp.exp(-jnp.abs(c))
            ce = jnp.maximum(c, 0.0) - c * t + _log1p_poly(e)
            inv = 1.0 / (1.0 + e)
            p = jnp.where(c >= 0, inv, e * inv)
            pt = jnp.where(t > 0.5, p, 1.0 - p)
            om = 1.0 - pt
            aw = jnp.where(t > 0.5, 0.25, 0.75)
            acc_cls = acc_cls + om * om * aw * ce * cared

            v0 = dir_b[pl.ds(b32, 16)]
            v1 = dir_b[pl.ds(b32 + 16, 16)]
            d0 = jnp.where(lo8, _take(v0, e0_lo), _take(v1, e0_hi))
            d1 = jnp.where(lo8, _take(v0, e1_lo), _take(v1, e1_hi))
            off = rot - _floor(rot * _INV_2PI) * _TWO_PI
            tgt = jnp.clip(_floor(off * _INV_PI), 0.0, 1.0)
            lse = jnp.maximum(d0, d1) + _log1p_poly(jnp.exp(-jnp.abs(d0 - d1)))
            dsel = jnp.where(tgt > 0.5, d1, d0)
            acc_dir = acc_dir + (lse - dsel) * pos

            acc_np = acc_np + pos
            return (acc_loc, acc_cls, acc_dir, acc_np)

        return body

    zero = jnp.zeros((16,), jnp.float32)
    accs = (zero, zero, zero, zero)

    _start(0, bufs0, sem0)

    def outer(i, accs):
        c0 = 2 * i
        _start(c0 + 1, bufs1, sem1)
        _wait(c0, bufs0, sem0)
        accs = lax.fori_loop(0, _NGROUP, _compute(bufs0), accs)

        @pl.when(c0 + 2 < _NCHUNK)
        def _():
            _start(c0 + 2, bufs0, sem0)

        _wait(c0 + 1, bufs1, sem1)
        accs = lax.fori_loop(0, _NGROUP, _compute(bufs1), accs)
        return accs

    accs = lax.fori_loop(0, _NCHUNK // 2, outer, accs)

    out_scr[0] = accs[0]
    out_scr[1] = accs[1]
    out_scr[2] = accs[2]
    out_scr[3] = accs[3]
    pltpu.sync_copy(out_scr, out_hbm.at[wid])


def kernel(box_preds, cls_preds, dir_cls_preds, reg_targets, anchors, labels):
    mesh = plsc.VectorSubcoreMesh(core_axis_name="c", subcore_axis_name="s")
    f32 = jnp.float32
    call = functools.partial(
        pl.kernel, mesh=mesh,
        out_type=jax.ShapeDtypeStruct((_NTEC, 4, 16), f32),
        scratch_types=(
            [pltpu.VMEM((_CHUNK * 7,), f32), pltpu.VMEM((_CHUNK * 7,), f32),
             pltpu.VMEM((_CHUNK * 7,), f32), pltpu.VMEM((_CHUNK,), f32),
             pltpu.VMEM((_CHUNK * 2,), f32), pltpu.VMEM((_CHUNK,), jnp.int32)]
            * 2
            + [pltpu.VMEM((4, 16), f32),
               pltpu.SemaphoreType.DMA, pltpu.SemaphoreType.DMA]),
    )(_sc_body)
    part = call(box_preds.reshape(-1), cls_preds.reshape(-1),
                dir_cls_preds.reshape(-1), reg_targets.reshape(-1),
                anchors.reshape(-1), labels.reshape(-1))
    ps = part.sum(-1).reshape(_B, _QPB, 4).sum(1)
    norm = jnp.maximum(ps[:, 3], 1.0)
    return ((2.0 * ps[:, 0] + ps[:, 1] + 0.2 * ps[:, 2]) / norm).sum() / _B


# speedup vs baseline: 25.4604x; 9.9926x over previous
import functools

import jax
import jax.numpy as jnp
from jax import lax
from jax.experimental import pallas as pl
from jax.experimental.pallas import tpu as pltpu
from jax.experimental.pallas import tpu_sc as plsc

_B = 8
_A = 70400
_NTILE = _A // 128
_NTEC = 32
_TPT = 17

_R = 1.0 / 9.0
_HR = 0.5 / 9.0
_TWO_PI = 6.283185307179586
_INV_2PI = 1.0 / _TWO_PI
_INV_PI = 1.0 / 3.141592653589793

_S = (0.9999845867744923, -0.16663258204297338, 0.008312382933816597,
      -0.00019316182196016648, 2.173210068101922e-06)
_L = (5.621958997561607e-07, 0.9999574870751128, -0.4992065685483709,
      0.3269731000160856, -0.22283625833236495, 0.13076503250835383,
      -0.05262485136951819, 0.01011908292798416)


def _floor(x):
    f = x.astype(jnp.int32).astype(jnp.float32)
    return jnp.where(f > x, f - 1.0, f)


def _log1p_poly(u):
    r = jnp.full_like(u, _L[7])
    for c in (_L[6], _L[5], _L[4], _L[3], _L[2], _L[1], _L[0]):
        r = r * u + c
    return r


def _sin_poly(x):
    n = _floor(x * _INV_2PI + 0.5)
    y = x - n * _TWO_PI
    y2 = y * y
    r = jnp.full_like(y, _S[4])
    for c in (_S[3], _S[2], _S[1], _S[0]):
        r = r * y2 + c
    return y * r


def _sc_body(box_hbm, cls_hbm, dir_hbm, reg_hbm, anc_hbm, lab_hbm, out_hbm,
             bx0, rg0, an0, cl0, dr0, lb0,
             bx1, rg1, an1, cl1, dr1, lb1,
             out_scr, sem0, sem1):
    cid = lax.axis_index("c")
    sid = lax.axis_index("s")
    wid = cid * 16 + sid
    t0 = wid * _TPT
    t_tail = _NTEC * _TPT + jnp.minimum(wid, 5)

    bufs0 = (bx0, rg0, an0, cl0, dr0, lb0)
    bufs1 = (bx1, rg1, an1, cl1, dr1, lb1)

    def _copies(t, bufs, sem):
        bx, rg, an, cl, dr, lb = bufs
        return (
            pltpu.make_async_copy(box_hbm.at[:, t, :, :], bx, sem),
            pltpu.make_async_copy(reg_hbm.at[:, t, :, :], rg, sem),
            pltpu.make_async_copy(anc_hbm.at[6, t, :, :], an, sem),
            pltpu.make_async_copy(cls_hbm.at[:, t, 0, :], cl, sem),
            pltpu.make_async_copy(dir_hbm.at[:, t, :, :], dr, sem),
            pltpu.make_async_copy(lab_hbm.at[t, :, :], lb, sem),
        )

    def _start(t, bufs, sem):
        for c in _copies(t, bufs, sem):
            c.start()

    def _wait(t, bufs, sem):
        for c in _copies(t, bufs, sem):
            c.wait()

    def _row_body(bufs, b):
        bx, rg, an, cl, dr, lb = bufs

        def body(g, accs):
            acc_loc, acc_cls, acc_dir, acc_np = accs
            s = g * 16

            lab = lb[b, pl.ds(s, 16)]
            pos = jnp.where(lab > 0, 1.0, 0.0)
            cared = jnp.where(lab >= 0, 1.0, 0.0)

            loc = None
            for c in range(6):
                d = bx[c, b, pl.ds(s, 16)] - rg[c, b, pl.ds(s, 16)]
                ad = jnp.abs(d)
                sl1 = jnp.where(ad < _R, 4.5 * d * d, ad - _HR)
                loc = sl1 if loc is None else loc + sl1
            r6 = rg[6, b, pl.ds(s, 16)]
            d6 = _sin_poly(bx[6, b, pl.ds(s, 16)] - r6)
            ad6 = jnp.abs(d6)
            loc = loc + jnp.where(ad6 < _R, 4.5 * d6 * d6, ad6 - _HR)
            acc_loc = acc_loc + loc * pos

            cv = cl[b, pl.ds(s, 16)]
            t = jnp.where(lab == 1, 1.0, 0.0)
            e = jnp.exp(-jnp.abs(cv))
            ce = jnp.maximum(cv, 0.0) - cv * t + _log1p_poly(e)
            inv = 1.0 / (1.0 + e)
            p = jnp.where(cv >= 0, inv, e * inv)
            pt = jnp.where(t > 0.5, p, 1.0 - p)
            om = 1.0 - pt
            aw = jnp.where(t > 0.5, 0.25, 0.75)
            acc_cls = acc_cls + om * om * aw * ce * cared

            rot = r6 + an[b, pl.ds(s, 16)]
            off = rot - _floor(rot * _INV_2PI) * _TWO_PI
            tgt = jnp.clip(_floor(off * _INV_PI), 0.0, 1.0)
            d0 = dr[b, 0, pl.ds(s, 16)]
            d1 = dr[b, 1, pl.ds(s, 16)]
            lse = jnp.maximum(d0, d1) + _log1p_poly(jnp.exp(-jnp.abs(d0 - d1)))
            dsel = jnp.where(tgt > 0.5, d1, d0)
            acc_dir = acc_dir + (lse - dsel) * pos

            acc_np = acc_np + pos
            return (acc_loc, acc_cls, acc_dir, acc_np)

        return body

    def _compute(bufs, accs):
        return tuple(
            lax.fori_loop(0, 8, _row_body(bufs, b), accs[b]) for b in range(8))

    zero = jnp.zeros((16,), jnp.float32)
    zaccs = tuple((zero, zero, zero, zero) for _ in range(_B))

    _start(t0, bufs0, sem0)

    def outer(i, accs):
        c0 = 2 * i
        _start(t0 + c0 + 1, bufs1, sem1)
        _wait(t0 + c0, bufs0, sem0)
        accs = _compute(bufs0, accs)

        @pl.when(c0 + 2 < _TPT)
        def _():
            _start(t0 + c0 + 2, bufs0, sem0)

        _wait(t0 + c0 + 1, bufs1, sem1)
        accs = _compute(bufs1, accs)
        return accs

    accs = lax.fori_loop(0, _TPT // 2, outer, zaccs)

    _start(t_tail, bufs1, sem1)
    _wait(t0 + _TPT - 1, bufs0, sem0)
    accs = _compute(bufs0, accs)
    _wait(t_tail, bufs1, sem1)
    tail = _compute(bufs1, zaccs)
    vmask = jnp.where(wid < 6, 1.0, 0.0)

    for b in range(_B):
        for j in range(4):
            out_scr[b * 4 + j] = accs[b][j] + tail[b][j] * vmask
    pltpu.sync_copy(out_scr, out_hbm.at[wid])


def kernel(box_preds, cls_preds, dir_cls_preds, reg_targets, anchors, labels):
    mesh = plsc.VectorSubcoreMesh(core_axis_name="c", subcore_axis_name="s")
    f32 = jnp.float32
    call = functools.partial(
        pl.kernel, mesh=mesh,
        out_type=jax.ShapeDtypeStruct((_NTEC, _B * 4, 16), f32),
        scratch_types=(
            [pltpu.VMEM((7, 8, 128), f32), pltpu.VMEM((7, 8, 128), f32),
             pltpu.VMEM((8, 128), f32), pltpu.VMEM((8, 128), f32),
             pltpu.VMEM((8, 2, 128), f32), pltpu.VMEM((8, 128), jnp.int32)]
            * 2
            + [pltpu.VMEM((_B * 4, 16), f32),
               pltpu.SemaphoreType.DMA, pltpu.SemaphoreType.DMA]),
    )(_sc_body)
    box_v = box_preds.transpose(2, 0, 1).reshape(7, 8, _NTILE, 128)
    box_v = box_v.transpose(0, 2, 1, 3)
    reg_v = reg_targets.transpose(2, 0, 1).reshape(7, 8, _NTILE, 128)
    reg_v = reg_v.transpose(0, 2, 1, 3)
    anc_v = anchors.transpose(2, 0, 1).reshape(7, 8, _NTILE, 128)
    anc_v = anc_v.transpose(0, 2, 1, 3)
    cls_v = cls_preds.reshape(_B, _NTILE, 1, 128)
    dir_v = dir_cls_preds.reshape(_B, _NTILE, 128, 2).transpose(0, 1, 3, 2)
    lab_v = labels.reshape(_B, _NTILE, 128).transpose(1, 0, 2)
    part = call(box_v, cls_v, dir_v, reg_v, anc_v, lab_v)
    ps = part.sum(-1).reshape(_NTEC, _B, 4).sum(0)
    norm = jnp.maximum(ps[:, 3], 1.0)
    return ((2.0 * ps[:, 0] + ps[:, 1] + 0.2 * ps[:, 2]) / norm).sum() / _B
